# Initial kernel scaffold; baseline (speedup 1.0000x reference)
import jax
import jax.numpy as jnp
from jax.experimental import pallas as pl


def kernel(data, params, channel_transform, spatio_transform):
    raise NotImplementedError("write your pallas kernel here")


# speedup vs baseline: 35.6569x; 35.6569x over previous
import jax
import jax.numpy as jnp
from jax.experimental import pallas as pl
from jax.experimental.pallas import tpu as pltpu

_P = 5
_LANES = 2048


def _minmax_kernel(x_ref, c_ref, mn_ref, mx_ref):
    i = pl.program_id(0)
    x = x_ref[...] * c_ref[0, 0]
    bmn = jnp.min(x)
    bmx = jnp.max(x)

    @pl.when(i == 0)
    def _():
        mn_ref[0, 0] = bmn
        mx_ref[0, 0] = bmx

    @pl.when(i > 0)
    def _():
        mn_ref[0, 0] = jnp.minimum(mn_ref[0, 0], bmn)
        mx_ref[0, 0] = jnp.maximum(mx_ref[0, 0], bmx)


def _hist_kernel(x_ref, c_ref, mn_ref, mx_ref, cum_ref):
    i = pl.program_id(0)
    dmin = mn_ref[0, 0]
    dmax = mx_ref[0, 0]
    span = dmax - dmin
    width = jnp.where(span > 0, span / _P, 1.0)
    x = x_ref[...] * c_ref[0, 0]

    @pl.when(i == 0)
    def _():
        for k in range(_P - 1):
            cum_ref[0, k] = 0.0

    for k in range(1, _P):
        t = dmin + width * k
        s = jnp.sum((x < t).astype(jnp.float32))
        cum_ref[0, k - 1] = cum_ref[0, k - 1] + s


def _chain4(ind, vals):
    return jnp.where(
        ind == 1, vals[0],
        jnp.where(ind == 2, vals[1], jnp.where(ind == 3, vals[2], vals[3])))


def _chain5(b, vals):
    return jnp.where(
        b == 0, vals[0],
        jnp.where(b == 1, vals[1],
                  jnp.where(b == 2, vals[2], jnp.where(b == 3, vals[3], vals[4]))))


def _interp_frame(accum, p, q):
    cnt = sum((accum[j] < q).astype(jnp.int32) for j in range(_P))
    ind = jnp.clip(cnt, 1, _P - 1)
    x0 = _chain4(ind, accum[0:4])
    x1 = _chain4(ind, accum[1:5])
    y0 = _chain4(ind, p[0:4])
    y1 = _chain4(ind, p[1:5])
    slope = (y1 - y0) / (x1 - x0)
    return y0 + slope * (q - x0)


def _xform_kernel(x_ref, c_ref, s_ref, mn_ref, mx_ref, cum_ref, prm_ref, n_ref,
                  o_ref):
    c = c_ref[0, 0]
    st = s_ref[0, 0]
    dmin = mn_ref[0, 0]
    dmax = mx_ref[0, 0]
    span = dmax - dmin
    width = jnp.where(span > 0, span / _P, 1.0)
    n = n_ref[0, 0]

    accum = [cum_ref[0, k] * (4.0 / n) for k in range(_P - 1)] + [
        jnp.float32(4.0)]
    grid = [dmin + width * (k + 0.5) for k in range(_P)]
    p_t = [prm_ref[0, k] for k in range(_P)]
    p_v = [prm_ref[1, k] for k in range(_P)]

    f = [_interp_frame(accum, p_t, jnp.float32(q)) for q in range(_P)]
    v = [_interp_frame(accum, p_v, jnp.float32(q)) for q in range(_P)]

    slope_seg = [(accum[j] - accum[j - 1]) / (grid[j] - grid[j - 1])
                 for j in range(1, _P)]
    a_seg = [accum[j - 1] - slope_seg[j - 1] * grid[j - 1] for j in range(1, _P)]

    f_base = f
    f_delta = [f[min(b + 1, _P - 1)] - f[b] for b in range(_P)]
    v_base = v
    v_delta = [v[min(b + 1, _P - 1)] - v[b] for b in range(_P)]

    x = x_ref[...] * c
    cnt = sum((x > grid[j]).astype(jnp.int32) for j in range(_P))
    ind = jnp.clip(cnt, 1, _P - 1)
    index = _chain4(ind, a_seg) + _chain4(ind, slope_seg) * x

    begin = jnp.clip(jnp.floor(index), 0.0, float(_P - 1))
    b = begin.astype(jnp.int32)
    pos = index - begin
    theta = _chain5(b, f_base) + pos * _chain5(b, f_delta)
    velo = _chain5(b, v_base) + pos * _chain5(b, v_delta)

    ds = velo * 0.01
    o_ref[...] = (x * (1.0 + ds * jnp.sin(theta)) + ds * jnp.cos(theta)) * st


def kernel(data, params, channel_transform, spatio_transform):
    shape = data.shape
    n = data.size
    rows = n // _LANES
    br = 512 if rows % 512 == 0 else rows
    nb = rows // br
    x2 = data.reshape(rows, _LANES)
    c = channel_transform.reshape(1, 1)
    s = spatio_transform.reshape(1, 1)
    prm = params.reshape(2, _P)
    nf = jnp.full((1, 1), float(n), dtype=jnp.float32)

    smem11 = pl.BlockSpec((1, 1), lambda i: (0, 0), memory_space=pltpu.SMEM)
    xspec = pl.BlockSpec((br, _LANES), lambda i: (i, 0))

    mn, mx = pl.pallas_call(
        _minmax_kernel,
        grid=(nb,),
        in_specs=[xspec, smem11],
        out_specs=[smem11, smem11],
        out_shape=[jax.ShapeDtypeStruct((1, 1), jnp.float32)] * 2,
    )(x2, c)

    cum = pl.pallas_call(
        _hist_kernel,
        grid=(nb,),
        in_specs=[xspec, smem11, smem11, smem11],
        out_specs=pl.BlockSpec((1, _P - 1), lambda i: (0, 0),
                               memory_space=pltpu.SMEM),
        out_shape=jax.ShapeDtypeStruct((1, _P - 1), jnp.float32),
    )(x2, c, mn, mx)

    out = pl.pallas_call(
        _xform_kernel,
        grid=(nb,),
        in_specs=[
            xspec, smem11, smem11, smem11, smem11,
            pl.BlockSpec((1, _P - 1), lambda i: (0, 0),
                         memory_space=pltpu.SMEM),
            pl.BlockSpec((2, _P), lambda i: (0, 0), memory_space=pltpu.SMEM),
            smem11,
        ],
        out_specs=xspec,
        out_shape=jax.ShapeDtypeStruct((rows, _LANES), jnp.float32),
    )(x2, c, s, mn, mx, cum, prm, nf)

    return out.reshape(shape)
